# baseline (speedup 1.0000x reference)
import jax
import jax.numpy as jnp
from jax import lax
from jax.experimental import pallas as pl
from jax.experimental.pallas import tpu as pltpu
from jax.experimental.pallas import tpu_sc as plsc

_B = 8192
_IN = 3072
_H = 2048
_L = 64
_K = 1024

_BM = 512
_HALF = _B // 2
_HB = _HALF // _BM

_NW = 32
_BPW = _HALF // _NW
_LP = 128

_NT = (((1,), (1,)), ((), ()))


def _encoder_math(x_ref, w1_ref, b1_ref, w2_ref, b2_ref, et_ref, ze_ref,
                  idx_ref):
    h = jnp.maximum(
        lax.dot_general(x_ref[...], w1_ref[...], _NT,
                        preferred_element_type=jnp.float32) + b1_ref[...], 0.0)
    z = lax.dot_general(h, w2_ref[...], _NT,
                        preferred_element_type=jnp.float32) + b2_ref[...]
    ze_ref[...] = z
    et = et_ref[...]
    scores = jnp.dot(z, et, preferred_element_type=jnp.float32)
    z2 = jnp.sum(z * z, axis=1, keepdims=True)
    e2 = jnp.sum(et * et, axis=0, keepdims=True)
    d2 = z2 + e2 - 2.0 * scores
    dist = jnp.sqrt(jnp.maximum(d2, 1e-12))
    m = jnp.min(dist, axis=1, keepdims=True)
    ii = lax.broadcasted_iota(jnp.int32, dist.shape, 1)
    idx_ref[...] = jnp.min(jnp.where(dist == m, ii, jnp.int32(_K)), axis=1,
                           keepdims=True)


def _encoder_body0(x_ref, w1_ref, b1_ref, w2_ref, b2_ref, et_ref, ze_ref,
                   idx_ref):
    _encoder_math(x_ref, w1_ref, b1_ref, w2_ref, b2_ref, et_ref, ze_ref,
                  idx_ref)


def _encoder_body1(x_ref, w1_ref, b1_ref, w2_ref, b2_ref, et_ref, ze_any,
                   ze_ref, idx_ref):
    _encoder_math(x_ref, w1_ref, b1_ref, w2_ref, b2_ref, et_ref, ze_ref,
                  idx_ref)


def _decoder_math(zq_ref, w3_ref, b3_ref, w4_ref, b4_ref, xr_ref, zqo_ref):
    zq = zq_ref[...][:, :_L]
    zqo_ref[...] = zq
    h = jnp.maximum(
        lax.dot_general(zq, w3_ref[...], _NT,
                        preferred_element_type=jnp.float32) + b3_ref[...], 0.0)
    xr_ref[...] = jax.nn.sigmoid(
        lax.dot_general(h, w4_ref[...], _NT,
                        preferred_element_type=jnp.float32) + b4_ref[...])


def _decoder_body0(zq_ref, w3_ref, b3_ref, w4_ref, b4_ref, xr_any, zqo_any,
                   xr_ref, zqo_ref):
    _decoder_math(zq_ref, w3_ref, b3_ref, w4_ref, b4_ref, xr_ref, zqo_ref)


def _decoder_onehot_body(idx_ref, emb_ref, w3_ref, b3_ref, w4_ref, b4_ref,
                         xr_ref, zqo_ref):
    ii = lax.broadcasted_iota(jnp.int32, (_BM, _K), 1)
    oh = (ii == idx_ref[...]).astype(jnp.float32)
    zq = lax.dot_general(oh, emb_ref[...], (((1,), (0,)), ((), ())),
                         precision=lax.Precision.HIGHEST,
                         preferred_element_type=jnp.float32)
    zqo_ref[...] = zq
    h = jnp.maximum(
        lax.dot_general(zq, w3_ref[...], _NT,
                        preferred_element_type=jnp.float32) + b3_ref[...], 0.0)
    xr_ref[...] = jax.nn.sigmoid(
        lax.dot_general(h, w4_ref[...], _NT,
                        preferred_element_type=jnp.float32) + b4_ref[...])


def _full(shape):
    return pl.BlockSpec(shape, lambda i: (0, 0))


def _encoder_call(x, W1, b1, W2, b2, embT, half, ze_buf=None):
    off = half * _HB
    base_specs = [
        pl.BlockSpec((_BM, _IN), lambda i: (i + off, 0)),
        _full((_H, _IN)),
        _full((1, _H)),
        _full((_L, _H)),
        _full((1, _L)),
        _full((_L, _K)),
    ]
    args = [x, W1, b1, W2, b2, embT]
    if half == 0:
        body, aliases = _encoder_body0, {}
    else:
        body, aliases = _encoder_body1, {6: 0}
        base_specs.append(pl.BlockSpec(memory_space=pl.ANY))
        args.append(ze_buf)
    return pl.pallas_call(
        body,
        grid=(_HB,),
        in_specs=base_specs,
        out_specs=(
            pl.BlockSpec((_BM, _L), lambda i: (i + off, 0)),
            pl.BlockSpec((_BM, 1), lambda i: (i, 0)),
        ),
        out_shape=(
            jax.ShapeDtypeStruct((_B, _L), jnp.float32),
            jax.ShapeDtypeStruct((_HALF, 1), jnp.int32),
        ),
        input_output_aliases=aliases,
        compiler_params=pltpu.CompilerParams(
            dimension_semantics=("arbitrary",)),
    )(*args)


_DEC_OUT_SHAPE = (
    jax.ShapeDtypeStruct((_B, _IN), jnp.float32),
    jax.ShapeDtypeStruct((_B, _L), jnp.float32),
)


def _decoder_call0(zq_half, W3, b3, W4, b4, xr_buf, zqo_buf):
    return pl.pallas_call(
        _decoder_body0,
        grid=(_HB,),
        in_specs=[
            pl.BlockSpec((_BM, _LP), lambda i: (i, 0)),
            _full((_H, _L)),
            _full((1, _H)),
            _full((_IN, _H)),
            _full((1, _IN)),
            pl.BlockSpec(memory_space=pl.ANY),
            pl.BlockSpec(memory_space=pl.ANY),
        ],
        out_specs=(
            pl.BlockSpec((_BM, _IN), lambda i: (i, 0)),
            pl.BlockSpec((_BM, _L), lambda i: (i, 0)),
        ),
        out_shape=_DEC_OUT_SHAPE,
        input_output_aliases={5: 0, 6: 1},
        compiler_params=pltpu.CompilerParams(
            dimension_semantics=("arbitrary",)),
    )(zq_half, W3, b3, W4, b4, xr_buf, zqo_buf)


def _decoder_call1(idx1, emb, W3, b3, W4, b4):
    return pl.pallas_call(
        _decoder_onehot_body,
        grid=(_HB,),
        in_specs=[
            pl.BlockSpec((_BM, 1), lambda i: (i, 0)),
            _full((_K, _L)),
            _full((_H, _L)),
            _full((1, _H)),
            _full((_IN, _H)),
            _full((1, _IN)),
        ],
        out_specs=(
            pl.BlockSpec((_BM, _IN), lambda i: (i + _HB, 0)),
            pl.BlockSpec((_BM, _L), lambda i: (i + _HB, 0)),
        ),
        out_shape=_DEC_OUT_SHAPE,
        compiler_params=pltpu.CompilerParams(
            dimension_semantics=("arbitrary",)),
    )(idx1, emb, W3, b3, W4, b4)


def _sc_gather_kernel(table_hbm, idx_hbm, out_hbm, idx_v, rows_v, sem):
    wid = lax.axis_index("s") * 2 + lax.axis_index("c")
    pltpu.sync_copy(idx_hbm.at[pl.ds(wid, 1)], idx_v)
    pltpu.async_copy(table_hbm.at[idx_v.at[0]], rows_v, sem).wait()
    pltpu.sync_copy(rows_v, out_hbm.at[pl.ds(wid * _BPW, _BPW)])


@jax.jit
def _sc_gather(table_padded, idx2d):
    mesh = plsc.VectorSubcoreMesh(core_axis_name="c", subcore_axis_name="s")
    return pl.kernel(
        _sc_gather_kernel,
        mesh=mesh,
        out_type=jax.ShapeDtypeStruct((_HALF, _LP), jnp.float32),
        scratch_types=[
            pltpu.VMEM((1, _BPW), jnp.int32),
            pltpu.VMEM((_BPW, _LP), jnp.float32),
            pltpu.SemaphoreType.DMA,
        ],
    )(table_padded, idx2d)


def kernel(x, W1, b1, W2, b2, W3, b3, W4, b4, embedding):
    embT = embedding.T
    table_padded = jnp.pad(embedding, ((0, 0), (0, _LP - _L)))
    b1r = b1.reshape(1, _H)
    b2r = b2.reshape(1, _L)
    b3r = b3.reshape(1, _H)
    b4r = b4.reshape(1, _IN)

    ze_p, idx0 = _encoder_call(x, W1, b1r, W2, b2r, embT, 0)
    zq0 = _sc_gather(table_padded, idx0.reshape(_NW, _BPW))
    z_e, idx1 = _encoder_call(x, W1, b1r, W2, b2r, embT, 1, ze_buf=ze_p)
    xr_p, zqo_p = _decoder_call1(idx1, embedding, W3, b3r, W4, b4r)
    x_recon, z_q = _decoder_call0(zq0, W3, b3r, W4, b4r, xr_p, zqo_p)
    return (x_recon, z_e, z_q)


# speedup vs baseline: 1.2392x; 1.2392x over previous
import jax
import jax.numpy as jnp
from jax import lax
from jax.experimental import pallas as pl
from jax.experimental.pallas import tpu as pltpu
from jax.experimental.pallas import tpu_sc as plsc

_B = 8192
_IN = 3072
_H = 2048
_L = 64
_K = 1024

_BM = 512
_HALF = _B // 2
_HB = _HALF // _BM

_NW = 32
_BPW = _HALF // _NW
_LP = 128

_NT = (((1,), (1,)), ((), ()))


def _encoder_math(x_ref, w1_ref, b1_ref, w2_ref, b2_ref, et_ref, ze_ref,
                  idx_ref):
    h = jnp.maximum(
        lax.dot_general(x_ref[...], w1_ref[...], _NT,
                        preferred_element_type=jnp.float32) + b1_ref[...], 0.0)
    z = lax.dot_general(h, w2_ref[...], _NT,
                        preferred_element_type=jnp.float32) + b2_ref[...]
    ze_ref[...] = z
    et = et_ref[...]
    scores = jnp.dot(z, et, preferred_element_type=jnp.float32)
    z2 = jnp.sum(z * z, axis=1, keepdims=True)
    e2 = jnp.sum(et * et, axis=0, keepdims=True)
    d2 = z2 + e2 - 2.0 * scores
    dist = jnp.sqrt(jnp.maximum(d2, 1e-12))
    m = jnp.min(dist, axis=1, keepdims=True)
    ii = lax.broadcasted_iota(jnp.int32, dist.shape, 1)
    idx_ref[...] = jnp.min(jnp.where(dist == m, ii, jnp.int32(_K)), axis=1,
                           keepdims=True)


def _encoder_body0(x_ref, w1_ref, b1_ref, w2_ref, b2_ref, et_ref, ze_ref,
                   idx_ref):
    _encoder_math(x_ref, w1_ref, b1_ref, w2_ref, b2_ref, et_ref, ze_ref,
                  idx_ref)


def _encoder_body1(x_ref, w1_ref, b1_ref, w2_ref, b2_ref, et_ref, ze_any,
                   ze_ref, idx_ref):
    _encoder_math(x_ref, w1_ref, b1_ref, w2_ref, b2_ref, et_ref, ze_ref,
                  idx_ref)


def _decoder_math(zq_ref, w3_ref, b3_ref, w4_ref, b4_ref, xr_ref, zqo_ref):
    zq = zq_ref[...][:, :_L]
    zqo_ref[...] = zq
    h = jnp.maximum(
        lax.dot_general(zq, w3_ref[...], _NT,
                        preferred_element_type=jnp.float32) + b3_ref[...], 0.0)
    xr_ref[...] = jax.nn.sigmoid(
        lax.dot_general(h, w4_ref[...], _NT,
                        preferred_element_type=jnp.float32) + b4_ref[...])


def _decoder_body0(zq_ref, w3_ref, b3_ref, w4_ref, b4_ref, xr_ref, zqo_ref):
    _decoder_math(zq_ref, w3_ref, b3_ref, w4_ref, b4_ref, xr_ref, zqo_ref)


def _decoder_body1(zq_ref, w3_ref, b3_ref, w4_ref, b4_ref, xr_any, zqo_any,
                   xr_ref, zqo_ref):
    _decoder_math(zq_ref, w3_ref, b3_ref, w4_ref, b4_ref, xr_ref, zqo_ref)


def _full(shape):
    return pl.BlockSpec(shape, lambda i: (0, 0))


def _encoder_call(x, W1, b1, W2, b2, embT, half, ze_buf=None):
    off = half * _HB
    base_specs = [
        pl.BlockSpec((_BM, _IN), lambda i: (i + off, 0)),
        _full((_H, _IN)),
        _full((1, _H)),
        _full((_L, _H)),
        _full((1, _L)),
        _full((_L, _K)),
    ]
    args = [x, W1, b1, W2, b2, embT]
    if half == 0:
        body, aliases = _encoder_body0, {}
    else:
        body, aliases = _encoder_body1, {6: 0}
        base_specs.append(pl.BlockSpec(memory_space=pl.ANY))
        args.append(ze_buf)
    return pl.pallas_call(
        body,
        grid=(_HB,),
        in_specs=base_specs,
        out_specs=(
            pl.BlockSpec((_BM, _L), lambda i: (i + off, 0)),
            pl.BlockSpec((_BM, 1), lambda i: (i, 0)),
        ),
        out_shape=(
            jax.ShapeDtypeStruct((_B, _L), jnp.float32),
            jax.ShapeDtypeStruct((_HALF, 1), jnp.int32),
        ),
        input_output_aliases=aliases,
        compiler_params=pltpu.CompilerParams(
            dimension_semantics=("arbitrary",)),
    )(*args)


_DEC_OUT_SHAPE = (
    jax.ShapeDtypeStruct((_B, _IN), jnp.float32),
    jax.ShapeDtypeStruct((_B, _L), jnp.float32),
)


def _decoder_call(zq_half, W3, b3, W4, b4, half, xr_buf=None, zqo_buf=None):
    off = half * _HB
    base_specs = [
        pl.BlockSpec((_BM, _LP), lambda i: (i, 0)),
        _full((_H, _L)),
        _full((1, _H)),
        _full((_IN, _H)),
        _full((1, _IN)),
    ]
    args = [zq_half, W3, b3, W4, b4]
    if half == 0:
        body, aliases = _decoder_body0, {}
    else:
        body, aliases = _decoder_body1, {5: 0, 6: 1}
        base_specs.append(pl.BlockSpec(memory_space=pl.ANY))
        base_specs.append(pl.BlockSpec(memory_space=pl.ANY))
        args.append(xr_buf)
        args.append(zqo_buf)
    return pl.pallas_call(
        body,
        grid=(_HB,),
        in_specs=base_specs,
        out_specs=(
            pl.BlockSpec((_BM, _IN), lambda i: (i + off, 0)),
            pl.BlockSpec((_BM, _L), lambda i: (i + off, 0)),
        ),
        out_shape=_DEC_OUT_SHAPE,
        input_output_aliases=aliases,
        compiler_params=pltpu.CompilerParams(
            dimension_semantics=("arbitrary",)),
    )(*args)


def _sc_gather_kernel(table_hbm, idx_hbm, out_hbm, tab_sh, idx_v, rows_v, sem):
    cid = lax.axis_index("c")
    sid = lax.axis_index("s")
    wid = sid * 2 + cid

    @pl.when(sid == 0)
    def _stage():
        pltpu.sync_copy(table_hbm, tab_sh)

    plsc.subcore_barrier()
    pltpu.sync_copy(idx_hbm.at[pl.ds(wid, 1)], idx_v)
    pltpu.async_copy(tab_sh.at[idx_v.at[0]], rows_v, sem).wait()
    pltpu.sync_copy(rows_v, out_hbm.at[pl.ds(wid * _BPW, _BPW)])


@jax.jit
def _sc_gather(table_padded, idx2d):
    mesh = plsc.VectorSubcoreMesh(core_axis_name="c", subcore_axis_name="s")
    return pl.kernel(
        _sc_gather_kernel,
        mesh=mesh,
        out_type=jax.ShapeDtypeStruct((_HALF, _LP), jnp.float32),
        scratch_types=[
            pltpu.VMEM_SHARED((_K, _LP), jnp.float32),
            pltpu.VMEM((1, _BPW), jnp.int32),
            pltpu.VMEM((_BPW, _LP), jnp.float32),
            pltpu.SemaphoreType.DMA,
        ],
    )(table_padded, idx2d)


def kernel(x, W1, b1, W2, b2, W3, b3, W4, b4, embedding):
    embT = embedding.T
    table_padded = jnp.pad(embedding, ((0, 0), (0, _LP - _L)))
    b1r = b1.reshape(1, _H)
    b2r = b2.reshape(1, _L)
    b3r = b3.reshape(1, _H)
    b4r = b4.reshape(1, _IN)

    ze_p, idx0 = _encoder_call(x, W1, b1r, W2, b2r, embT, 0)
    zq0 = _sc_gather(table_padded, idx0.reshape(_NW, _BPW))
    z_e, idx1 = _encoder_call(x, W1, b1r, W2, b2r, embT, 1, ze_buf=ze_p)
    zq1 = _sc_gather(table_padded, idx1.reshape(_NW, _BPW))
    xr_p, zqo_p = _decoder_call(zq0, W3, b3r, W4, b4r, 0)
    x_recon, z_q = _decoder_call(zq1, W3, b3r, W4, b4r, 1, xr_buf=xr_p,
                                 zqo_buf=zqo_p)
    return (x_recon, z_e, z_q)
